# baseline (speedup 1.0000x reference)
import functools

import jax
import jax.numpy as jnp
from jax import lax
from jax.experimental import pallas as pl
from jax.experimental.pallas import tpu as pltpu
from jax.experimental.pallas import tpu_sc as plsc

N_EXPERTS = 8
D_MODEL = 1024
D_PACK = D_MODEL // 2
N_TOKENS = 4096

BLK = 256
NBLK = N_TOKENS // BLK + 8
NPAD = NBLK * BLK

NW = 32
TPW = N_TOKENS // NW
GCH = 32
NCH = TPW // GCH

_mesh = plsc.VectorSubcoreMesh(core_axis_name="core", subcore_axis_name="subcore")


SCH = 32
NSC = TPW // SCH


@functools.partial(
    pl.kernel,
    out_type=jax.ShapeDtypeStruct((NPAD, D_MODEL), jnp.float32),
    mesh=_mesh,
    scratch_types=[
        pltpu.VMEM((NSC, SCH), jnp.int32),
        pltpu.VMEM((SCH, D_MODEL), jnp.float32),
        pltpu.VMEM((SCH, D_MODEL), jnp.float32),
        pltpu.SemaphoreType.DMA,
        pltpu.SemaphoreType.DMA,
    ],
)
def _sc_scatter(x_hbm, idx_hbm, o_hbm, idx_v, buf_a, buf_b, sem_a, sem_b):
    wid = lax.axis_index("subcore") * 2 + lax.axis_index("core")
    base = wid * TPW
    bufs = (buf_a, buf_b)
    sems = (sem_a, sem_b)
    for c in range(NSC):
        pltpu.sync_copy(idx_hbm.at[pl.ds(base + c * SCH, SCH)], idx_v.at[c])
    cps = [pltpu.async_copy(x_hbm.at[pl.ds(base + c * SCH, SCH)],
                            bufs[c % 2], sems[c % 2])
           for c in range(2)]
    for c in range(NSC):
        cps[c].wait()
        pltpu.sync_copy(bufs[c % 2], o_hbm.at[idx_v.at[c]])
        if c + 2 < NSC:
            cps.append(pltpu.async_copy(x_hbm.at[pl.ds(base + (c + 2) * SCH, SCH)],
                                        bufs[c % 2], sems[c % 2]))


@functools.partial(
    pl.kernel,
    out_type=jax.ShapeDtypeStruct((N_TOKENS, D_MODEL), jnp.float32),
    mesh=_mesh,
    scratch_types=[
        pltpu.VMEM((NCH, GCH), jnp.int32),
        pltpu.VMEM((GCH, D_MODEL), jnp.float32),
        pltpu.VMEM((GCH, D_MODEL), jnp.float32),
        pltpu.SemaphoreType.DMA,
        pltpu.SemaphoreType.DMA,
    ],
)
def _sc_gather(src_hbm, idx_hbm, o_hbm, idx_v, buf_a, buf_b, sem_a, sem_b):
    wid = lax.axis_index("subcore") * 2 + lax.axis_index("core")
    base = wid * TPW
    bufs = (buf_a, buf_b)
    sems = (sem_a, sem_b)
    for c in range(NCH):
        pltpu.sync_copy(idx_hbm.at[pl.ds(base + c * GCH, GCH)], idx_v.at[c])
    cps = [pltpu.async_copy(src_hbm.at[idx_v.at[c]], bufs[c % 2], sems[c % 2])
           for c in range(2)]
    for c in range(NCH):
        cps[c].wait()
        pltpu.sync_copy(bufs[c % 2], o_hbm.at[pl.ds(base + c * GCH, GCH)])
        if c + 2 < NCH:
            cps.append(pltpu.async_copy(src_hbm.at[idx_v.at[c + 2]],
                                        bufs[c % 2], sems[c % 2]))


def _tc_expert_matmul(x_pad, W, b, expert_of_block, nblk_total):
    def mm_kernel(e_ref, v_ref, x_ref, w_ref, b_ref, o_ref):
        @pl.when(pl.program_id(0) < v_ref[0])
        def _():
            o_ref[...] = (jnp.dot(x_ref[...].astype(jnp.bfloat16), w_ref[0],
                                  preferred_element_type=jnp.float32)
                          + b_ref[0])

    grid_spec = pltpu.PrefetchScalarGridSpec(
        num_scalar_prefetch=2,
        grid=(NBLK,),
        in_specs=[
            pl.BlockSpec((BLK, D_MODEL),
                         lambda i, e, v: (jnp.minimum(i, v[0] - 1), 0)),
            pl.BlockSpec((1, D_MODEL, D_MODEL), lambda i, e, v: (e[i], 0, 0)),
            pl.BlockSpec((1, 1, D_MODEL), lambda i, e, v: (e[i], 0, 0)),
        ],
        out_specs=pl.BlockSpec((BLK, D_MODEL),
                               lambda i, e, v: (jnp.minimum(i, v[0] - 1), 0)),
    )
    return pl.pallas_call(
        mm_kernel,
        grid_spec=grid_spec,
        out_shape=jax.ShapeDtypeStruct((NPAD, D_MODEL), jnp.float32),
    )(expert_of_block, nblk_total, x_pad, W, b.reshape(N_EXPERTS, 1, D_MODEL))


def kernel(x, groups, W, b):
    g = groups[:, 0].astype(jnp.int32)

    onehot_i = (g[:, None] == jnp.arange(N_EXPERTS, dtype=jnp.int32)[None, :]
                ).astype(jnp.int32)
    cum = jnp.cumsum(onehot_i, axis=0)
    counts = cum[-1]
    nblk = (counts + BLK - 1) // BLK
    cum_nblk = jnp.cumsum(nblk)
    nblk_total = cum_nblk[-1:].astype(jnp.int32)
    pad_start = (cum_nblk - nblk) * BLK
    padpos = jnp.sum(onehot_i * (pad_start[None, :] + cum - 1),
                     axis=1).astype(jnp.int32)
    expert_of_block = jnp.minimum(
        jnp.sum((jnp.arange(NBLK, dtype=jnp.int32)[:, None]
                 >= cum_nblk[None, :]).astype(jnp.int32), axis=1),
        N_EXPERTS - 1).astype(jnp.int32)

    Wb = W.astype(jnp.bfloat16)
    x_pad = _sc_scatter(x, padpos)
    out_pad = _tc_expert_matmul(x_pad, Wb, b, expert_of_block, nblk_total)
    return _sc_gather(out_pad, padpos)


# speedup vs baseline: 1.0255x; 1.0255x over previous
import functools

import jax
import jax.numpy as jnp
from jax import lax
from jax.experimental import pallas as pl
from jax.experimental.pallas import tpu as pltpu
from jax.experimental.pallas import tpu_sc as plsc

N_EXPERTS = 8
D_MODEL = 1024
N_TOKENS = 4096

BLK = 256
NBLK = N_TOKENS // BLK + 8
NPAD = NBLK * BLK

NW = 32
TPW = N_TOKENS // NW
SCH = 64
NSC = TPW // SCH

_mesh = plsc.VectorSubcoreMesh(core_axis_name="core", subcore_axis_name="subcore")


@functools.partial(
    pl.kernel,
    out_type=jax.ShapeDtypeStruct((NPAD, D_MODEL), jnp.float32),
    mesh=_mesh,
    scratch_types=[
        pltpu.VMEM((NSC, SCH), jnp.int32),
        pltpu.VMEM((SCH, D_MODEL), jnp.float32),
    ],
)
def _sc_scatter(x_hbm, idx_hbm, o_hbm, idx_v, rows_v):
    wid = lax.axis_index("subcore") * 2 + lax.axis_index("core")
    base = wid * TPW
    for c in range(NSC):
        pltpu.sync_copy(idx_hbm.at[pl.ds(base + c * SCH, SCH)], idx_v.at[c])
        pltpu.sync_copy(x_hbm.at[pl.ds(base + c * SCH, SCH)], rows_v)
        pltpu.sync_copy(rows_v, o_hbm.at[idx_v.at[c]])


@functools.partial(
    pl.kernel,
    out_type=jax.ShapeDtypeStruct((N_TOKENS, D_MODEL), jnp.float32),
    mesh=_mesh,
    scratch_types=[
        pltpu.VMEM((NSC, SCH), jnp.int32),
        pltpu.VMEM((SCH, D_MODEL), jnp.float32),
    ],
)
def _sc_gather(src_hbm, idx_hbm, o_hbm, idx_v, rows_v):
    wid = lax.axis_index("subcore") * 2 + lax.axis_index("core")
    base = wid * TPW
    for c in range(NSC):
        pltpu.sync_copy(idx_hbm.at[pl.ds(base + c * SCH, SCH)], idx_v.at[c])
        pltpu.sync_copy(src_hbm.at[idx_v.at[c]], rows_v)
        pltpu.sync_copy(rows_v, o_hbm.at[pl.ds(base + c * SCH, SCH)])


def _tc_expert_matmul(x_pad, W, b, expert_of_block, nblk_total):
    def mm_kernel(e_ref, v_ref, x_ref, w_ref, b_ref, o_ref):
        @pl.when(pl.program_id(0) < v_ref[0])
        def _():
            e = e_ref[pl.program_id(0)]
            o_ref[...] = (jnp.dot(x_ref[...].astype(jnp.bfloat16), w_ref[e],
                                  preferred_element_type=jnp.float32)
                          + b_ref[e])

    grid_spec = pltpu.PrefetchScalarGridSpec(
        num_scalar_prefetch=2,
        grid=(NBLK,),
        in_specs=[
            pl.BlockSpec((BLK, D_MODEL),
                         lambda i, e, v: (jnp.minimum(i, v[0] - 1), 0)),
            pl.BlockSpec((N_EXPERTS, D_MODEL, D_MODEL), lambda i, e, v: (0, 0, 0)),
            pl.BlockSpec((N_EXPERTS, 1, D_MODEL), lambda i, e, v: (0, 0, 0)),
        ],
        out_specs=pl.BlockSpec((BLK, D_MODEL),
                               lambda i, e, v: (jnp.minimum(i, v[0] - 1), 0)),
    )
    return pl.pallas_call(
        mm_kernel,
        grid_spec=grid_spec,
        out_shape=jax.ShapeDtypeStruct((NPAD, D_MODEL), jnp.float32),
    )(expert_of_block, nblk_total, x_pad, W, b.reshape(N_EXPERTS, 1, D_MODEL))


def kernel(x, groups, W, b):
    g = groups[:, 0].astype(jnp.int32)

    onehot_i = (g[:, None] == jnp.arange(N_EXPERTS, dtype=jnp.int32)[None, :]
                ).astype(jnp.int32)
    cum = jnp.cumsum(onehot_i, axis=0)
    counts = cum[-1]
    nblk = (counts + BLK - 1) // BLK
    cum_nblk = jnp.cumsum(nblk)
    nblk_total = cum_nblk[-1:].astype(jnp.int32)
    pad_start = (cum_nblk - nblk) * BLK
    padpos = jnp.sum(onehot_i * (pad_start[None, :] + cum - 1),
                     axis=1).astype(jnp.int32)
    expert_of_block = jnp.minimum(
        jnp.sum((jnp.arange(NBLK, dtype=jnp.int32)[:, None]
                 >= cum_nblk[None, :]).astype(jnp.int32), axis=1),
        N_EXPERTS - 1).astype(jnp.int32)

    Wb = W.astype(jnp.bfloat16)
    x_pad = _sc_scatter(x, padpos)
    out_pad = _tc_expert_matmul(x_pad, Wb, b, expert_of_block, nblk_total)
    return _sc_gather(out_pad, padpos)
